# Initial kernel scaffold; baseline (speedup 1.0000x reference)
import jax
import jax.numpy as jnp
from jax.experimental import pallas as pl


def kernel(x, edge_index, Wr0, br0, Wo0, Wr1, br1, Wo1, Wr2, br2, Wo2):
    raise NotImplementedError("write your pallas kernel here")


# speedup vs baseline: 5.0882x; 5.0882x over previous
import functools

import jax
import jax.numpy as jnp
from jax import lax
from jax.experimental import pallas as pl
from jax.experimental.pallas import tpu as pltpu
from jax.experimental.pallas import tpu_sc as plsc

N = 10000
E = 320000
D = 128

NC = 2
NS = 16
NW = NC * NS

EPW = E // NW
CH = 80
NCHUNK = EPW // CH
NP = 10240
ROWS_PT = NP // NS
WB = 128


def _sc_aggregate_body(h_hbm, src_hbm, dst_hbm, out_hbm,
                       idx_src, idx_dst, rows, wb_buf, acc, sem):
    cid = lax.axis_index("c")
    sid = lax.axis_index("s")
    wid = cid * NS + sid

    def _zero_row(i):
        for j in range(D // 16):
            wb_buf[i, pl.ds(j * 16, 16)] = jnp.zeros((16,), jnp.float32)
    pl.loop(0, WB)(_zero_row)
    row0 = sid * ROWS_PT
    for j in range(ROWS_PT // WB):
        pltpu.sync_copy(wb_buf, acc.at[pl.ds(row0 + j * WB, WB), :])
    plsc.subcore_barrier()

    ebase = wid * EPW

    def _chunk(k):
        off = ebase + k * CH
        pltpu.sync_copy(src_hbm.at[pl.ds(off, CH)], idx_src)
        pltpu.sync_copy(dst_hbm.at[pl.ds(off, CH)], idx_dst)
        pltpu.async_copy(h_hbm.at[idx_src], rows, sem).wait()
        pltpu.sync_copy(rows, acc.at[idx_dst], add=True)
    pl.loop(0, NCHUNK)(_chunk)

    plsc.subcore_barrier()

    for j in range(ROWS_PT // WB):
        r = row0 + j * WB
        pltpu.sync_copy(acc.at[pl.ds(r, WB), :], wb_buf)
        pltpu.sync_copy(wb_buf, out_hbm.at[cid, pl.ds(r, WB), :])


_sc_aggregate = pl.kernel(
    _sc_aggregate_body,
    out_type=jax.ShapeDtypeStruct((NC, NP, D), jnp.float32),
    mesh=plsc.VectorSubcoreMesh(core_axis_name="c", subcore_axis_name="s"),
    scratch_types=[
        pltpu.VMEM((CH,), jnp.int32),
        pltpu.VMEM((CH,), jnp.int32),
        pltpu.VMEM((CH, D), jnp.float32),
        pltpu.VMEM((WB, D), jnp.float32),
        pltpu.VMEM_SHARED((NP, D), jnp.float32),
        pltpu.SemaphoreType.DMA,
    ],
)


RB = 1000


def _dense_body(do_relu, p_ref, h_ref, wr_ref, br_ref, wo_ref, o_ref):
    agg = p_ref[0] + p_ref[1]
    o = (jnp.dot(agg, wr_ref[...], preferred_element_type=jnp.float32)
         + br_ref[...]
         + jnp.dot(h_ref[...], wo_ref[...], preferred_element_type=jnp.float32))
    if do_relu:
        o = jnp.maximum(o, 0.0)
    o_ref[...] = o


def _dense(p, h, wr, br, wo, do_relu):
    return pl.pallas_call(
        functools.partial(_dense_body, do_relu),
        grid=(N // RB,),
        in_specs=[
            pl.BlockSpec((NC, RB, D), lambda i: (0, i, 0)),
            pl.BlockSpec((RB, D), lambda i: (i, 0)),
            pl.BlockSpec((D, D), lambda i: (0, 0)),
            pl.BlockSpec((D,), lambda i: (0,)),
            pl.BlockSpec((D, D), lambda i: (0, 0)),
        ],
        out_specs=pl.BlockSpec((RB, D), lambda i: (i, 0)),
        out_shape=jax.ShapeDtypeStruct((N, D), jnp.float32),
    )(p, h, wr, br, wo)


def kernel(x, edge_index, Wr0, br0, Wo0, Wr1, br1, Wo1, Wr2, br2, Wo2):
    src = edge_index[0]
    dst = edge_index[1]
    h = x
    for i, (wr, br, wo) in enumerate(
            ((Wr0, br0, Wo0), (Wr1, br1, Wo1), (Wr2, br2, Wo2))):
        p = _sc_aggregate(h, src, dst)
        h = _dense(p, h, wr, br, wo, do_relu=(i < 2))
    return h


# speedup vs baseline: 13.3307x; 2.6199x over previous
import functools

import jax
import jax.numpy as jnp
from jax import lax
from jax.experimental import pallas as pl
from jax.experimental.pallas import tpu as pltpu
from jax.experimental.pallas import tpu_sc as plsc

N = 10000
E = 320000
D = 128

NC = 2
NS = 16
NW = NC * NS

EPW = E // NW
CH = 40
NCHUNK = EPW // CH
NBUF = 4
RBI = 8
GROUPS = (NCHUNK - 10) // RBI
NZC = N // CH


def _sc_aggregate_body(h_hbm, eidx_hbm, out_hbm, ring, rows, acc,
                       sem_z, sem_x, sem_g):
    cid = lax.axis_index("c")
    sid = lax.axis_index("s")
    wid = cid * NS + sid

    for j in range(RBI):
        pltpu.async_copy(eidx_hbm.at[wid, j], ring.at[j], sem_x.at[j])

    def _zero_row(i):
        for j in range(D // 16):
            rows[0, i, pl.ds(j * 16, 16)] = jnp.zeros((16,), jnp.float32)
    pl.loop(0, CH)(_zero_row)
    for j in range(NZC // NS):
        c = sid + NS * j
        pltpu.async_copy(rows.at[0], acc.at[pl.ds(c * CH, CH), :], sem_z)

    @pl.when(sid < NZC % NS)
    def _():
        pltpu.async_copy(rows.at[0],
                         acc.at[pl.ds((sid + NS * (NZC // NS)) * CH, CH), :],
                         sem_z)
    for j in range(NZC // NS):
        pltpu.make_async_copy(rows.at[0], acc.at[pl.ds(0, CH), :],
                              sem_z).wait()

    @pl.when(sid < NZC % NS)
    def _():
        pltpu.make_async_copy(rows.at[0], acc.at[pl.ds(0, CH), :],
                              sem_z).wait()
    plsc.subcore_barrier()

    for j in range(NBUF):
        pltpu.make_async_copy(eidx_hbm.at[wid, j], ring.at[j],
                              sem_x.at[j]).wait()
        pltpu.async_copy(h_hbm.at[ring.at[j, 0]], rows.at[j], sem_g.at[j])

    def _step(k, b, slot, do_idx, do_gather):
        pltpu.make_async_copy(h_hbm.at[pl.ds(0, CH)], rows.at[b],
                              sem_g.at[b]).wait()
        pltpu.sync_copy(rows.at[b], acc.at[ring.at[slot, 1]], add=True)
        if do_idx:
            pltpu.async_copy(eidx_hbm.at[wid, k + RBI], ring.at[slot],
                             sem_x.at[slot])
        if do_gather:
            s2 = (slot + NBUF) % RBI
            pltpu.make_async_copy(eidx_hbm.at[wid, 0], ring.at[s2],
                                  sem_x.at[s2]).wait()
            pltpu.async_copy(h_hbm.at[ring.at[s2, 0]], rows.at[b],
                             sem_g.at[b])

    def _group(g):
        for j in range(RBI):
            _step(g * RBI + j, j % NBUF, j, True, True)
    pl.loop(0, GROUPS)(_group)

    for k in range(GROUPS * RBI, NCHUNK):
        _step(k, k % NBUF, k % RBI, k + RBI < NCHUNK, k + NBUF < NCHUNK)

    plsc.subcore_barrier()

    for j in range(NZC // NS):
        c = (sid + NS * j) * CH
        pltpu.sync_copy(acc.at[pl.ds(c, CH), :],
                        out_hbm.at[cid, pl.ds(c, CH), :])

    @pl.when(sid < NZC % NS)
    def _():
        c = (sid + NS * (NZC // NS)) * CH
        pltpu.sync_copy(acc.at[pl.ds(c, CH), :],
                        out_hbm.at[cid, pl.ds(c, CH), :])


_sc_aggregate = pl.kernel(
    _sc_aggregate_body,
    out_type=jax.ShapeDtypeStruct((NC, N, D), jnp.float32),
    mesh=plsc.VectorSubcoreMesh(core_axis_name="c", subcore_axis_name="s"),
    scratch_types=[
        pltpu.VMEM((RBI, 2, CH), jnp.int32),
        pltpu.VMEM((NBUF, CH, D), jnp.float32),
        pltpu.VMEM_SHARED((N, D), jnp.float32),
        pltpu.SemaphoreType.DMA,
        pltpu.SemaphoreType.DMA((RBI,)),
        pltpu.SemaphoreType.DMA((NBUF,)),
    ],
)


RB = 1000


def _dense_body(do_relu, p_ref, h_ref, wr_ref, br_ref, wo_ref, o_ref):
    agg = p_ref[0] + p_ref[1]
    o = (jnp.dot(agg, wr_ref[...], preferred_element_type=jnp.float32)
         + br_ref[...]
         + jnp.dot(h_ref[...], wo_ref[...], preferred_element_type=jnp.float32))
    if do_relu:
        o = jnp.maximum(o, 0.0)
    o_ref[...] = o


def _dense(p, h, wr, br, wo, do_relu):
    return pl.pallas_call(
        functools.partial(_dense_body, do_relu),
        grid=(N // RB,),
        in_specs=[
            pl.BlockSpec((NC, RB, D), lambda i: (0, i, 0)),
            pl.BlockSpec((RB, D), lambda i: (i, 0)),
            pl.BlockSpec((D, D), lambda i: (0, 0)),
            pl.BlockSpec((D,), lambda i: (0,)),
            pl.BlockSpec((D, D), lambda i: (0, 0)),
        ],
        out_specs=pl.BlockSpec((RB, D), lambda i: (i, 0)),
        out_shape=jax.ShapeDtypeStruct((N, D), jnp.float32),
    )(p, h, wr, br, wo)


def kernel(x, edge_index, Wr0, br0, Wo0, Wr1, br1, Wo1, Wr2, br2, Wo2):
    eidx = edge_index.reshape(2, NW, NCHUNK, CH).transpose(1, 2, 0, 3)
    h = x
    for i, (wr, br, wo) in enumerate(
            ((Wr0, br0, Wo0), (Wr1, br1, Wo1), (Wr2, br2, Wo2))):
        p = _sc_aggregate(h, eidx)
        h = _dense(p, h, wr, br, wo, do_relu=(i < 2))
    return h


# speedup vs baseline: 14.0821x; 1.0564x over previous
import functools

import jax
import jax.numpy as jnp
from jax import lax
from jax.experimental import pallas as pl
from jax.experimental.pallas import tpu as pltpu
from jax.experimental.pallas import tpu_sc as plsc

N = 10000
E = 320000
D = 128

NC = 2
NS = 16
NW = NC * NS

EPW = E // NW
CH = 40
NCHUNK = EPW // CH
NBUF = 4
RBI = 8
GROUPS = (NCHUNK - 10) // RBI
NZC = N // CH


def _sc_aggregate_body(h_hbm, eidx_hbm, out_hbm, ring, rows, acc,
                       sem_z, sem_x, sem_g):
    cid = lax.axis_index("c")
    sid = lax.axis_index("s")
    wid = cid * NS + sid

    for j in range(RBI):
        pltpu.async_copy(eidx_hbm.at[wid, j], ring.at[j], sem_x.at[j])

    def _zero_row(i):
        for j in range(D // 16):
            rows[0, i, pl.ds(j * 16, 16)] = jnp.zeros((16,), jnp.float32)
    pl.loop(0, CH)(_zero_row)
    for j in range(NZC // NS):
        c = sid + NS * j
        pltpu.async_copy(rows.at[0], acc.at[pl.ds(c * CH, CH), :], sem_z)

    @pl.when(sid < NZC % NS)
    def _():
        pltpu.async_copy(rows.at[0],
                         acc.at[pl.ds((sid + NS * (NZC // NS)) * CH, CH), :],
                         sem_z)
    for j in range(NZC // NS):
        pltpu.make_async_copy(rows.at[0], acc.at[pl.ds(0, CH), :],
                              sem_z).wait()

    @pl.when(sid < NZC % NS)
    def _():
        pltpu.make_async_copy(rows.at[0], acc.at[pl.ds(0, CH), :],
                              sem_z).wait()
    plsc.subcore_barrier()

    for j in range(NBUF):
        pltpu.make_async_copy(eidx_hbm.at[wid, j], ring.at[j],
                              sem_x.at[j]).wait()
        pltpu.async_copy(h_hbm.at[ring.at[j, 0]], rows.at[j], sem_g.at[j])

    def _step(k, b, slot, do_idx, do_gather):
        pltpu.make_async_copy(h_hbm.at[pl.ds(0, CH)], rows.at[b],
                              sem_g.at[b]).wait()
        pltpu.sync_copy(rows.at[b], acc.at[ring.at[slot, 1]], add=True)
        if do_idx:
            pltpu.async_copy(eidx_hbm.at[wid, k + RBI], ring.at[slot],
                             sem_x.at[slot])
        if do_gather:
            s2 = (slot + NBUF) % RBI
            pltpu.make_async_copy(eidx_hbm.at[wid, 0], ring.at[s2],
                                  sem_x.at[s2]).wait()
            pltpu.async_copy(h_hbm.at[ring.at[s2, 0]], rows.at[b],
                             sem_g.at[b])

    def _group(g):
        for j in range(RBI):
            _step(g * RBI + j, j % NBUF, j, True, True)
    pl.loop(0, GROUPS)(_group)

    for k in range(GROUPS * RBI, NCHUNK):
        _step(k, k % NBUF, k % RBI, k + RBI < NCHUNK, k + NBUF < NCHUNK)

    plsc.subcore_barrier()

    for j in range(NZC // NS):
        c = (sid + NS * j) * CH
        pltpu.async_copy(acc.at[pl.ds(c, CH), :],
                         out_hbm.at[cid, pl.ds(c, CH), :], sem_z)

    @pl.when(sid < NZC % NS)
    def _():
        c = (sid + NS * (NZC // NS)) * CH
        pltpu.async_copy(acc.at[pl.ds(c, CH), :],
                         out_hbm.at[cid, pl.ds(c, CH), :], sem_z)
    for j in range(NZC // NS):
        pltpu.make_async_copy(acc.at[pl.ds(0, CH), :],
                              out_hbm.at[cid, pl.ds(0, CH), :], sem_z).wait()

    @pl.when(sid < NZC % NS)
    def _():
        pltpu.make_async_copy(acc.at[pl.ds(0, CH), :],
                              out_hbm.at[cid, pl.ds(0, CH), :], sem_z).wait()


_sc_aggregate = pl.kernel(
    _sc_aggregate_body,
    out_type=jax.ShapeDtypeStruct((NC, N, D), jnp.float32),
    mesh=plsc.VectorSubcoreMesh(core_axis_name="c", subcore_axis_name="s"),
    scratch_types=[
        pltpu.VMEM((RBI, 2, CH), jnp.int32),
        pltpu.VMEM((NBUF, CH, D), jnp.float32),
        pltpu.VMEM_SHARED((N, D), jnp.float32),
        pltpu.SemaphoreType.DMA,
        pltpu.SemaphoreType.DMA((RBI,)),
        pltpu.SemaphoreType.DMA((NBUF,)),
    ],
)


RB = 2000


def _dense_body(do_relu, p_ref, h_ref, wr_ref, br_ref, wo_ref, o_ref):
    agg = p_ref[0] + p_ref[1]
    o = (jnp.dot(agg, wr_ref[...], preferred_element_type=jnp.float32)
         + br_ref[...]
         + jnp.dot(h_ref[...], wo_ref[...], preferred_element_type=jnp.float32))
    if do_relu:
        o = jnp.maximum(o, 0.0)
    o_ref[...] = o


def _dense(p, h, wr, br, wo, do_relu):
    return pl.pallas_call(
        functools.partial(_dense_body, do_relu),
        grid=(N // RB,),
        in_specs=[
            pl.BlockSpec((NC, RB, D), lambda i: (0, i, 0)),
            pl.BlockSpec((RB, D), lambda i: (i, 0)),
            pl.BlockSpec((D, D), lambda i: (0, 0)),
            pl.BlockSpec((D,), lambda i: (0,)),
            pl.BlockSpec((D, D), lambda i: (0, 0)),
        ],
        out_specs=pl.BlockSpec((RB, D), lambda i: (i, 0)),
        out_shape=jax.ShapeDtypeStruct((N, D), jnp.float32),
    )(p, h, wr, br, wo)


def kernel(x, edge_index, Wr0, br0, Wo0, Wr1, br1, Wo1, Wr2, br2, Wo2):
    eidx = edge_index.reshape(2, NW, NCHUNK, CH).transpose(1, 2, 0, 3)
    h = x
    for i, (wr, br, wo) in enumerate(
            ((Wr0, br0, Wo0), (Wr1, br1, Wo1), (Wr2, br2, Wo2))):
        p = _sc_aggregate(h, eidx)
        h = _dense(p, h, wr, br, wo, do_relu=(i < 2))
    return h
